# baseline (speedup 1.0000x reference)
import functools

import jax
import jax.numpy as jnp
from jax import lax
from jax.experimental import pallas as pl
from jax.experimental.pallas import tpu as pltpu

_TOTAL_LABELS = 81
_TOTAL_POS = 32
_TOTAL_NEG = 96

_INTERPRET = False

_RBLK = 256
_FBLK = _RBLK * _TOTAL_LABELS


def _stage_a(roi_ref, gt_ref, gl_ref, rpos_ref, rneg_ref,
             delta_ref, glab_ref, mpos_ref, mneg_ref):
    roi = roi_ref[0]
    gt = gt_ref[0]
    gl = gl_ref[0]
    m = gt.shape[0]
    blk = roi.shape[1]

    by1 = roi[0:1, :]
    bx1 = roi[1:2, :]
    by2 = roi[2:3, :]
    bx2 = roi[3:4, :]
    gy1 = gt[:, 0:1]
    gx1 = gt[:, 1:2]
    gy2 = gt[:, 2:3]
    gx2 = gt[:, 3:4]

    gt_area = (gy2 - gy1) * (gx2 - gx1)
    bbox_area = (by2 - by1) * (bx2 - bx1)
    x_top = jnp.maximum(bx1, gx1)
    y_top = jnp.maximum(by1, gy1)
    x_bot = jnp.minimum(bx2, gx2)
    y_bot = jnp.minimum(by2, gy2)
    inter = jnp.maximum(x_bot - x_top, 0.0) * jnp.maximum(y_bot - y_top, 0.0)
    union = bbox_area + gt_area - inter
    iou = inter / (union + 1e-9)

    merged = jnp.max(iou, axis=0, keepdims=True)
    sub = lax.broadcasted_iota(jnp.int32, (m, blk), 0)
    midx = jnp.min(jnp.where(iou == merged, sub, m), axis=0, keepdims=True)
    onehot = sub == midx

    def gsel(col):
        return jnp.max(jnp.where(onehot, col, jnp.zeros_like(col)),
                       axis=0, keepdims=True)

    gb_y1 = gsel(gy1)
    gb_x1 = gsel(gx1)
    gb_y2 = gsel(gy2)
    gb_x2 = gsel(gx2)
    glab = gsel(gl)

    bw = bx2 - bx1
    bh = by2 - by1
    bcx = bx1 + 0.5 * bw
    bcy = by1 + 0.5 * bh
    gw = gb_x2 - gb_x1
    gh = gb_y2 - gb_y1
    gcx = gb_x1 + 0.5 * gw
    gcy = gb_y1 + 0.5 * gh
    bw = jnp.where(bw == 0, 1e-3, bw)
    bh = jnp.where(bh == 0, 1e-3, bh)
    zw = gw == 0
    zh = gh == 0
    dx = jnp.where(zw, 0.0, (gcx - bcx) / bw)
    dy = jnp.where(zh, 0.0, (gcy - bcy) / bh)
    dw = jnp.where(zw, 0.0, jnp.log(jnp.where(zw, 1.0, gw) / bw))
    dh = jnp.where(zh, 0.0, jnp.log(jnp.where(zh, 1.0, gh) / bh))
    delta = jnp.concatenate(
        [dy / 0.1, dx / 0.1, dh / 0.2, dw / 0.2], axis=0)

    pos_c = merged > 0.5
    neg_c = jnp.logical_and(merged < 0.5, merged > 0.1)

    delta_ref[0] = delta
    glab_ref[0] = glab
    mpos_ref[0] = pos_c.astype(jnp.int32) * rpos_ref[0]
    mneg_ref[0] = neg_c.astype(jnp.int32) * rneg_ref[0]


def _solve_threshold(mult, count_sel, vmax):
    b, n = mult.shape
    lane = lax.broadcasted_iota(jnp.int32, (b, n), 1)
    lo = jnp.zeros((b, 1), jnp.int32)
    hi = jnp.full((b, 1), vmax, jnp.int32)
    for _ in range(11):
        mid = lo + (hi - lo) // 2
        cnt = jnp.sum((mult >= mid).astype(jnp.int32), axis=1, keepdims=True)
        pred = cnt >= count_sel
        lo = jnp.where(pred, mid, lo)
        hi = jnp.where(pred, hi, mid)
    t = lo
    g = jnp.sum((mult >= t + 1).astype(jnp.int32), axis=1, keepdims=True)
    k = count_sel - g
    eq = mult == t
    lo2 = jnp.full((b, 1), -1, jnp.int32)
    hi2 = jnp.full((b, 1), n - 1, jnp.int32)
    for _ in range(15):
        mid = lo2 + (hi2 - lo2) // 2
        p = jnp.sum(jnp.logical_and(eq, lane <= mid).astype(jnp.int32),
                    axis=1, keepdims=True)
        pred = p >= k
        hi2 = jnp.where(pred, mid, hi2)
        lo2 = jnp.where(pred, lo2, mid)
    return t, hi2


def _stage_b(mpos_ref, mneg_ref, tpos_ref, cpos_ref, tneg_ref, cneg_ref):
    tp, cp = _solve_threshold(mpos_ref[...], _TOTAL_POS, _TOTAL_POS * 10)
    tn, cn = _solve_threshold(mneg_ref[...], _TOTAL_NEG, _TOTAL_NEG * 10)
    tpos_ref[...] = tp[:, :, None]
    cpos_ref[...] = cp[:, :, None]
    tneg_ref[...] = tn[:, :, None]
    cneg_ref[...] = cn[:, :, None]


def _stage_c(delta_ref, glab_ref, mpos_ref, mneg_ref,
             tpos_ref, cpos_ref, tneg_ref, cneg_ref, slot_ref, rmat_ref,
             labels_ref, deltas_ref, *, nv):
    j = pl.program_id(1)
    gidx = j * _RBLK + lax.broadcasted_iota(jnp.int32, (1, _RBLK), 1)
    valid = gidx < nv
    mp = mpos_ref[0]
    mn = mneg_ref[0]
    tp = tpos_ref[0]
    cp = cpos_ref[0]
    tn = tneg_ref[0]
    cn = cneg_ref[0]
    pos = jnp.logical_and(
        mp > 0, jnp.logical_or(mp > tp, jnp.logical_and(mp == tp, gidx <= cp)))
    neg = jnp.logical_and(
        mn > 0, jnp.logical_or(mn > tn, jnp.logical_and(mn == tn, gidx <= cn)))
    pos = jnp.logical_and(pos, valid)
    neg = jnp.logical_and(neg, valid)

    glab = glab_ref[0]
    label_eff = jnp.where(pos, glab,
                          jnp.where(neg, jnp.int32(0), jnp.int32(-1)))
    lab_col = lax.broadcasted_iota(jnp.int32, (_TOTAL_LABELS, 1), 0)
    labels_ref[0] = (lab_col == label_eff).astype(jnp.float32)

    labm = jnp.where(pos, glab.astype(jnp.float32), -1.0)
    dm = jnp.where(pos, delta_ref[0], 0.0)
    mat = jnp.concatenate([labm, dm], axis=0)
    rep = jnp.dot(mat, rmat_ref[...],
                  preferred_element_type=jnp.float32)
    eqf = (rep[0:1, :] == slot_ref[0]).astype(jnp.float32)
    deltas_ref[0] = rep[1:5, :] * eqf


def kernel(roi_bboxes, gt_boxes, gt_labels):
    b, n = roi_bboxes.shape[0], roi_bboxes.shape[1]
    m = gt_boxes.shape[1]
    blk_a = 2048

    rpos = jax.random.randint(jax.random.key(42), (b, n), 1, _TOTAL_POS * 10)
    rneg = jax.random.randint(jax.random.key(43), (b, n), 1, _TOTAL_NEG * 10)
    rpos3 = rpos.astype(jnp.int32)[:, None, :]
    rneg3 = rneg.astype(jnp.int32)[:, None, :]
    roi_t = jnp.transpose(roi_bboxes, (0, 2, 1))
    gl2 = gt_labels[:, :, None]

    delta_t, glab_t, mpos_t, mneg_t = pl.pallas_call(
        _stage_a,
        grid=(b, pl.cdiv(n, blk_a)),
        in_specs=[
            pl.BlockSpec((1, 4, blk_a), lambda i, j: (i, 0, j)),
            pl.BlockSpec((1, m, 4), lambda i, j: (i, 0, 0)),
            pl.BlockSpec((1, m, 1), lambda i, j: (i, 0, 0)),
            pl.BlockSpec((1, 1, blk_a), lambda i, j: (i, 0, j)),
            pl.BlockSpec((1, 1, blk_a), lambda i, j: (i, 0, j)),
        ],
        out_specs=[
            pl.BlockSpec((1, 4, blk_a), lambda i, j: (i, 0, j)),
            pl.BlockSpec((1, 1, blk_a), lambda i, j: (i, 0, j)),
            pl.BlockSpec((1, 1, blk_a), lambda i, j: (i, 0, j)),
            pl.BlockSpec((1, 1, blk_a), lambda i, j: (i, 0, j)),
        ],
        out_shape=[
            jax.ShapeDtypeStruct((b, 4, n), jnp.float32),
            jax.ShapeDtypeStruct((b, 1, n), jnp.int32),
            jax.ShapeDtypeStruct((b, 1, n), jnp.int32),
            jax.ShapeDtypeStruct((b, 1, n), jnp.int32),
        ],
        compiler_params=pltpu.CompilerParams(
            dimension_semantics=("parallel", "parallel")),
        interpret=_INTERPRET,
    )(roi_t, gt_boxes, gl2, rpos3, rneg3)

    tpos, cpos, tneg, cneg = pl.pallas_call(
        _stage_b,
        out_shape=[jax.ShapeDtypeStruct((b, 1, 1), jnp.int32)] * 4,
        interpret=_INTERPRET,
    )(mpos_t.reshape(b, n), mneg_t.reshape(b, n))

    fidx = jnp.arange(_FBLK, dtype=jnp.int32)
    slotf = (fidx % _TOTAL_LABELS).astype(jnp.float32)[None, None, :]
    rmat = (fidx // _TOTAL_LABELS == jnp.arange(_RBLK)[:, None]
            ).astype(jnp.float32)

    labels_t, deltas_t = pl.pallas_call(
        functools.partial(_stage_c, nv=n),
        grid=(b, pl.cdiv(n, _RBLK)),
        in_specs=[
            pl.BlockSpec((1, 4, _RBLK), lambda i, j: (i, 0, j)),
            pl.BlockSpec((1, 1, _RBLK), lambda i, j: (i, 0, j)),
            pl.BlockSpec((1, 1, _RBLK), lambda i, j: (i, 0, j)),
            pl.BlockSpec((1, 1, _RBLK), lambda i, j: (i, 0, j)),
            pl.BlockSpec((1, 1, 1), lambda i, j: (i, 0, 0)),
            pl.BlockSpec((1, 1, 1), lambda i, j: (i, 0, 0)),
            pl.BlockSpec((1, 1, 1), lambda i, j: (i, 0, 0)),
            pl.BlockSpec((1, 1, 1), lambda i, j: (i, 0, 0)),
            pl.BlockSpec((1, 1, _FBLK), lambda i, j: (0, 0, 0)),
            pl.BlockSpec((_RBLK, _FBLK), lambda i, j: (0, 0)),
        ],
        out_specs=[
            pl.BlockSpec((1, _TOTAL_LABELS, _RBLK), lambda i, j: (i, 0, j)),
            pl.BlockSpec((1, 4, _FBLK), lambda i, j: (i, 0, j)),
        ],
        out_shape=[
            jax.ShapeDtypeStruct((b, _TOTAL_LABELS, n), jnp.float32),
            jax.ShapeDtypeStruct((b, 4, n * _TOTAL_LABELS), jnp.float32),
        ],
        compiler_params=pltpu.CompilerParams(
            dimension_semantics=("parallel", "parallel")),
        interpret=_INTERPRET,
    )(delta_t, glab_t, mpos_t, mneg_t, tpos, cpos, tneg, cneg, slotf, rmat)

    roi_bbox_deltas = jnp.transpose(deltas_t, (0, 2, 1))
    roi_bbox_labels = jnp.transpose(labels_t, (0, 2, 1))
    return roi_bbox_deltas, roi_bbox_labels


# speedup vs baseline: 2.4432x; 2.4432x over previous
import functools

import jax
import jax.numpy as jnp
from jax import lax
from jax.experimental import pallas as pl
from jax.experimental.pallas import tpu as pltpu

_TOTAL_LABELS = 81
_TOTAL_POS = 32
_TOTAL_NEG = 96

_INTERPRET = False

_RBLK = 128
_GRP = 16
_CBLK = _RBLK * _GRP
_FBLK = _RBLK * _TOTAL_LABELS


def _stage_a(roi_ref, gt_ref, gl_ref, rpos_ref, rneg_ref,
             delta_ref, glab_ref, mpos_ref, mneg_ref):
    roi = roi_ref[0]
    gt = gt_ref[0]
    gl = gl_ref[0]
    m = gt.shape[0]
    blk = roi.shape[1]

    by1 = roi[0:1, :]
    bx1 = roi[1:2, :]
    by2 = roi[2:3, :]
    bx2 = roi[3:4, :]
    gy1 = gt[:, 0:1]
    gx1 = gt[:, 1:2]
    gy2 = gt[:, 2:3]
    gx2 = gt[:, 3:4]

    gt_area = (gy2 - gy1) * (gx2 - gx1)
    bbox_area = (by2 - by1) * (bx2 - bx1)
    x_top = jnp.maximum(bx1, gx1)
    y_top = jnp.maximum(by1, gy1)
    x_bot = jnp.minimum(bx2, gx2)
    y_bot = jnp.minimum(by2, gy2)
    inter = jnp.maximum(x_bot - x_top, 0.0) * jnp.maximum(y_bot - y_top, 0.0)
    union = bbox_area + gt_area - inter
    iou = inter / (union + 1e-9)

    merged = jnp.max(iou, axis=0, keepdims=True)
    sub = lax.broadcasted_iota(jnp.int32, (m, blk), 0)
    midx = jnp.min(jnp.where(iou == merged, sub, m), axis=0, keepdims=True)
    onehot = sub == midx

    def gsel(col):
        return jnp.max(jnp.where(onehot, col, jnp.zeros_like(col)),
                       axis=0, keepdims=True)

    gb_y1 = gsel(gy1)
    gb_x1 = gsel(gx1)
    gb_y2 = gsel(gy2)
    gb_x2 = gsel(gx2)
    glab = gsel(gl)

    bw = bx2 - bx1
    bh = by2 - by1
    bcx = bx1 + 0.5 * bw
    bcy = by1 + 0.5 * bh
    gw = gb_x2 - gb_x1
    gh = gb_y2 - gb_y1
    gcx = gb_x1 + 0.5 * gw
    gcy = gb_y1 + 0.5 * gh
    bw = jnp.where(bw == 0, 1e-3, bw)
    bh = jnp.where(bh == 0, 1e-3, bh)
    zw = gw == 0
    zh = gh == 0
    dx = jnp.where(zw, 0.0, (gcx - bcx) / bw)
    dy = jnp.where(zh, 0.0, (gcy - bcy) / bh)
    dw = jnp.where(zw, 0.0, jnp.log(jnp.where(zw, 1.0, gw) / bw))
    dh = jnp.where(zh, 0.0, jnp.log(jnp.where(zh, 1.0, gh) / bh))
    delta = jnp.concatenate(
        [dy / 0.1, dx / 0.1, dh / 0.2, dw / 0.2], axis=0)

    pos_c = merged > 0.5
    neg_c = jnp.logical_and(merged < 0.5, merged > 0.1)

    delta_ref[0] = delta
    glab_ref[0] = glab
    mpos_ref[0] = pos_c.astype(jnp.int32) * rpos_ref[0]
    mneg_ref[0] = neg_c.astype(jnp.int32) * rneg_ref[0]


def _solve_threshold(mult, count_sel, vmax):
    b, n = mult.shape
    lane = lax.broadcasted_iota(jnp.int32, (b, n), 1)
    lo = jnp.zeros((b, 1), jnp.int32)
    hi = jnp.full((b, 1), vmax, jnp.int32)
    for _ in range(11):
        mid = lo + (hi - lo) // 2
        cnt = jnp.sum((mult >= mid).astype(jnp.int32), axis=1, keepdims=True)
        pred = cnt >= count_sel
        lo = jnp.where(pred, mid, lo)
        hi = jnp.where(pred, hi, mid)
    t = lo
    g = jnp.sum((mult >= t + 1).astype(jnp.int32), axis=1, keepdims=True)
    k = count_sel - g
    eq = mult == t
    lo2 = jnp.full((b, 1), -1, jnp.int32)
    hi2 = jnp.full((b, 1), n - 1, jnp.int32)
    for _ in range(15):
        mid = lo2 + (hi2 - lo2) // 2
        p = jnp.sum(jnp.logical_and(eq, lane <= mid).astype(jnp.int32),
                    axis=1, keepdims=True)
        pred = p >= k
        hi2 = jnp.where(pred, mid, hi2)
        lo2 = jnp.where(pred, lo2, mid)
    return t, hi2


def _stage_b(mpos_ref, mneg_ref, tpos_ref, cpos_ref, tneg_ref, cneg_ref):
    tp, cp = _solve_threshold(mpos_ref[...], _TOTAL_POS, _TOTAL_POS * 10)
    tn, cn = _solve_threshold(mneg_ref[...], _TOTAL_NEG, _TOTAL_NEG * 10)
    tpos_ref[...] = tp[:, :, None]
    cpos_ref[...] = cp[:, :, None]
    tneg_ref[...] = tn[:, :, None]
    cneg_ref[...] = cn[:, :, None]


def _stage_c(delta_ref, glab_ref, mpos_ref, mneg_ref,
             tpos_ref, cpos_ref, tneg_ref, cneg_ref, slot_ref, rmat_ref,
             labels_ref, deltas_ref, *, nv):
    j = pl.program_id(1)
    gidx = j * _CBLK + lax.broadcasted_iota(jnp.int32, (1, _CBLK), 1)
    valid = gidx < nv
    mp = mpos_ref[0]
    mn = mneg_ref[0]
    tp = tpos_ref[0]
    cp = cpos_ref[0]
    tn = tneg_ref[0]
    cn = cneg_ref[0]
    pos = jnp.logical_and(
        mp > 0, jnp.logical_or(mp > tp, jnp.logical_and(mp == tp, gidx <= cp)))
    neg = jnp.logical_and(
        mn > 0, jnp.logical_or(mn > tn, jnp.logical_and(mn == tn, gidx <= cn)))
    pos = jnp.logical_and(pos, valid)
    neg = jnp.logical_and(neg, valid)

    glab = glab_ref[0]
    label_eff = jnp.where(pos, glab,
                          jnp.where(neg, jnp.int32(0), jnp.int32(-1)))
    lab_col = lax.broadcasted_iota(jnp.int32, (_TOTAL_LABELS, 1), 0)
    labels_ref[0] = (lab_col == label_eff).astype(jnp.float32)

    labm = jnp.where(pos, glab.astype(jnp.float32), -1.0)
    dm = jnp.where(pos, delta_ref[0], 0.0)
    mat = jnp.concatenate(
        [jnp.concatenate(
            [labm[:, g * _RBLK:(g + 1) * _RBLK],
             dm[:, g * _RBLK:(g + 1) * _RBLK]], axis=0)
         for g in range(_GRP)], axis=0)
    rep = jnp.dot(mat, rmat_ref[...],
                  preferred_element_type=jnp.float32)
    slot = slot_ref[0]
    pieces = []
    for g in range(_GRP):
        eqf = (rep[5 * g:5 * g + 1, :] == slot).astype(jnp.float32)
        pieces.append(rep[5 * g + 1:5 * g + 5, :] * eqf)
    deltas_ref[0] = jnp.concatenate(pieces, axis=1)


def kernel(roi_bboxes, gt_boxes, gt_labels):
    b, n = roi_bboxes.shape[0], roi_bboxes.shape[1]
    m = gt_boxes.shape[1]
    blk_a = 2048

    rpos = jax.random.randint(jax.random.key(42), (b, n), 1, _TOTAL_POS * 10)
    rneg = jax.random.randint(jax.random.key(43), (b, n), 1, _TOTAL_NEG * 10)
    rpos3 = rpos.astype(jnp.int32)[:, None, :]
    rneg3 = rneg.astype(jnp.int32)[:, None, :]
    roi_t = jnp.transpose(roi_bboxes, (0, 2, 1))
    gl2 = gt_labels[:, :, None]

    delta_t, glab_t, mpos_t, mneg_t = pl.pallas_call(
        _stage_a,
        grid=(b, pl.cdiv(n, blk_a)),
        in_specs=[
            pl.BlockSpec((1, 4, blk_a), lambda i, j: (i, 0, j)),
            pl.BlockSpec((1, m, 4), lambda i, j: (i, 0, 0)),
            pl.BlockSpec((1, m, 1), lambda i, j: (i, 0, 0)),
            pl.BlockSpec((1, 1, blk_a), lambda i, j: (i, 0, j)),
            pl.BlockSpec((1, 1, blk_a), lambda i, j: (i, 0, j)),
        ],
        out_specs=[
            pl.BlockSpec((1, 4, blk_a), lambda i, j: (i, 0, j)),
            pl.BlockSpec((1, 1, blk_a), lambda i, j: (i, 0, j)),
            pl.BlockSpec((1, 1, blk_a), lambda i, j: (i, 0, j)),
            pl.BlockSpec((1, 1, blk_a), lambda i, j: (i, 0, j)),
        ],
        out_shape=[
            jax.ShapeDtypeStruct((b, 4, n), jnp.float32),
            jax.ShapeDtypeStruct((b, 1, n), jnp.int32),
            jax.ShapeDtypeStruct((b, 1, n), jnp.int32),
            jax.ShapeDtypeStruct((b, 1, n), jnp.int32),
        ],
        compiler_params=pltpu.CompilerParams(
            dimension_semantics=("parallel", "parallel")),
        interpret=_INTERPRET,
    )(roi_t, gt_boxes, gl2, rpos3, rneg3)

    tpos, cpos, tneg, cneg = pl.pallas_call(
        _stage_b,
        out_shape=[jax.ShapeDtypeStruct((b, 1, 1), jnp.int32)] * 4,
        interpret=_INTERPRET,
    )(mpos_t.reshape(b, n), mneg_t.reshape(b, n))

    fidx = jnp.arange(_FBLK, dtype=jnp.int32)
    slotf = (fidx % _TOTAL_LABELS).astype(jnp.float32)[None, None, :]
    rmat = (fidx // _TOTAL_LABELS == jnp.arange(_RBLK)[:, None]
            ).astype(jnp.float32)

    labels_t, deltas_t = pl.pallas_call(
        functools.partial(_stage_c, nv=n),
        grid=(b, pl.cdiv(n, _CBLK)),
        in_specs=[
            pl.BlockSpec((1, 4, _CBLK), lambda i, j: (i, 0, j)),
            pl.BlockSpec((1, 1, _CBLK), lambda i, j: (i, 0, j)),
            pl.BlockSpec((1, 1, _CBLK), lambda i, j: (i, 0, j)),
            pl.BlockSpec((1, 1, _CBLK), lambda i, j: (i, 0, j)),
            pl.BlockSpec((1, 1, 1), lambda i, j: (i, 0, 0)),
            pl.BlockSpec((1, 1, 1), lambda i, j: (i, 0, 0)),
            pl.BlockSpec((1, 1, 1), lambda i, j: (i, 0, 0)),
            pl.BlockSpec((1, 1, 1), lambda i, j: (i, 0, 0)),
            pl.BlockSpec((1, 1, _FBLK), lambda i, j: (0, 0, 0)),
            pl.BlockSpec((_RBLK, _FBLK), lambda i, j: (0, 0)),
        ],
        out_specs=[
            pl.BlockSpec((1, _TOTAL_LABELS, _CBLK), lambda i, j: (i, 0, j)),
            pl.BlockSpec((1, 4, _GRP * _FBLK), lambda i, j: (i, 0, j)),
        ],
        out_shape=[
            jax.ShapeDtypeStruct((b, _TOTAL_LABELS, n), jnp.float32),
            jax.ShapeDtypeStruct((b, 4, n * _TOTAL_LABELS), jnp.float32),
        ],
        compiler_params=pltpu.CompilerParams(
            dimension_semantics=("parallel", "parallel")),
        interpret=_INTERPRET,
    )(delta_t, glab_t, mpos_t, mneg_t, tpos, cpos, tneg, cneg, slotf, rmat)

    roi_bbox_deltas = jnp.transpose(deltas_t, (0, 2, 1))
    roi_bbox_labels = jnp.transpose(labels_t, (0, 2, 1))
    return roi_bbox_deltas, roi_bbox_labels
